# Initial kernel scaffold; baseline (speedup 1.0000x reference)
import jax
import jax.numpy as jnp
from jax.experimental import pallas as pl


def kernel(coords, features, res, points_num):
    raise NotImplementedError("write your pallas kernel here")


# speedup vs baseline: 18.4946x; 18.4946x over previous
import functools

import jax
import jax.numpy as jnp
from jax import lax
from jax.experimental import pallas as pl
from jax.experimental.pallas import tpu as pltpu
from jax.experimental.pallas import tpu_sc as plsc

H = 48
W = 48
S = H * W
N = 1024
B = 16
K = 16
L = 16
WPB = 2
QPW = S // WPB
NG = QPW // L


def _sc_knn(px, py, fl, fh, pn):
    mesh = plsc.VectorSubcoreMesh(core_axis_name="c", subcore_axis_name="s")

    @functools.partial(
        pl.kernel,
        out_type=jax.ShapeDtypeStruct((B, WPB, QPW * 2), jnp.float32),
        mesh=mesh,
        compiler_params=pltpu.CompilerParams(needs_layout_passes=False),
        scratch_types=[
            pltpu.VMEM((N,), jnp.float32),
            pltpu.VMEM((N,), jnp.float32),
            pltpu.VMEM((N,), jnp.float32),
            pltpu.VMEM((N,), jnp.float32),
            pltpu.VMEM((B,), jnp.int32),
            pltpu.VMEM((QPW * 2,), jnp.float32),
        ],
    )
    def knn(px_hbm, py_hbm, fl_hbm, fh_hbm, pn_hbm, out_hbm,
            px_v, py_v, fl_v, fh_v, pn_v, out_v):
        wid = lax.axis_index("s") * 2 + lax.axis_index("c")
        b = wid // WPB
        half = wid % WPB
        pltpu.sync_copy(px_hbm.at[b], px_v)
        pltpu.sync_copy(py_hbm.at[b], py_v)
        pltpu.sync_copy(fl_hbm.at[b], fl_v)
        pltpu.sync_copy(fh_hbm.at[b], fh_v)
        pltpu.sync_copy(pn_hbm, pn_v)
        lanes = jnp.arange(L, dtype=jnp.int32)
        pn_b = plsc.load_gather(pn_v, [jnp.zeros((L,), jnp.int32) + b])
        small = pn_b < K
        qbase = half * QPW

        def group_body(g, carry):
            def q_body(i, acc):
                a0, a1 = acc
                q = qbase + g * L + i
                qx = (q // W).astype(jnp.float32)
                qy = (q % W).astype(jnp.float32)
                bk0 = jnp.full((L,), jnp.inf, jnp.float32)
                bv0 = jnp.full((L,), 2**30, jnp.int32)

                def chunk(t, c):
                    bk, bv = c
                    o = t * L
                    dx = px_v[pl.ds(o, L)] - qx
                    dy = py_v[pl.ds(o, L)] - qy
                    d = dx * dx + dy * dy
                    ds_, is_ = plsc.sort_key_val(d, o + lanes, descending=True)
                    take = (ds_ < bk) | ((ds_ == bk) & (is_ < bv))
                    nk = jnp.where(take, ds_, bk)
                    nv = jnp.where(take, is_, bv)
                    nk, nv = plsc.sort_key_val(nk, nv)
                    return (nk, nv)

                _, bv = lax.fori_loop(0, N // L, chunk, (bk0, bv0))
                bv = jnp.where(small, lanes, bv)
                f0 = plsc.load_gather(fl_v, [bv])
                f1 = plsc.load_gather(fh_v, [bv])
                s0 = jnp.sum(f0)
                s1 = jnp.sum(f1)
                c0 = jnp.sum(jnp.where(f0 != 0.0, 1.0, 0.0))
                c1 = jnp.sum(jnp.where(f1 != 0.0, 1.0, 0.0))
                c0 = jnp.where(c0 == 0.0, 1.0, c0)
                c1 = jnp.where(c1 == 0.0, 1.0, c1)
                av0 = jnp.broadcast_to(s0, (L,)) / jnp.broadcast_to(c0, (L,))
                av1 = jnp.broadcast_to(s1, (L,)) / jnp.broadcast_to(c1, (L,))
                sel = lanes == i
                a0 = jnp.where(sel, av0, a0)
                a1 = jnp.where(sel, av1, a1)
                return a0, a1

            z = jnp.zeros((L,), jnp.float32)
            a0, a1 = lax.fori_loop(0, L, q_body, (z, z))
            m = jnp.maximum(a0, a1)
            u0 = jnp.exp(a0 - m)
            u1 = jnp.exp(a1 - m)
            den = u0 + u1
            p0 = u0 / den
            p1 = u1 / den
            eq = p0 == p1
            p0 = jnp.where(eq, 1.0, p0)
            p1 = jnp.where(eq, 0.0, p1)
            i0 = g * (2 * L) + 2 * lanes
            plsc.store_scatter(out_v, [i0], p0)
            plsc.store_scatter(out_v, [i0 + 1], p1)
            return carry

        lax.fori_loop(0, NG, group_body, 0)
        pltpu.sync_copy(out_v, out_hbm.at[b, half])

    return knn(px, py, fl, fh, pn)


def kernel(coords, features, res, points_num):
    p = jnp.asarray(res, jnp.float32)
    cmax = jnp.max(coords, axis=-2, keepdims=True)
    cmin = jnp.min(coords, axis=-2, keepdims=True)
    center = (cmax + cmin) / 2
    scale = jnp.maximum(cmax - cmin, 1e-05) / 2
    cn = ((coords - center) / scale + 1) * 0.8 * p / 2 + 0.1 * p
    valid = jnp.arange(N)[None, :] < points_num[:, None]
    px = jnp.where(valid, cn[..., 0], 1e30)
    py = jnp.where(valid, cn[..., 1], 1e30)
    fl = jnp.minimum(features[..., 0], features[..., 1])
    fh = jnp.maximum(features[..., 0], features[..., 1])
    out = _sc_knn(px, py, fl, fh, points_num.astype(jnp.int32))
    return out.reshape(B, H, W, 2)


# speedup vs baseline: 58.3905x; 3.1572x over previous
import functools

import jax
import jax.numpy as jnp
from jax import lax
from jax.experimental import pallas as pl
from jax.experimental.pallas import tpu as pltpu
from jax.experimental.pallas import tpu_sc as plsc

H = 48
W = 48
S = H * W
N = 1024
B = 16
K = 16
L = 16
WPB = 2
QPW = S // WPB
NG = QPW // L


def _sc_knn(px, py, fl, fh, pn):
    mesh = plsc.VectorSubcoreMesh(core_axis_name="c", subcore_axis_name="s")

    @functools.partial(
        pl.kernel,
        out_type=jax.ShapeDtypeStruct((B, WPB, QPW * 2), jnp.float32),
        mesh=mesh,
        compiler_params=pltpu.CompilerParams(needs_layout_passes=False),
        scratch_types=[
            pltpu.VMEM((N,), jnp.float32),
            pltpu.VMEM((N,), jnp.float32),
            pltpu.VMEM((N,), jnp.float32),
            pltpu.VMEM((N,), jnp.float32),
            pltpu.VMEM((B,), jnp.int32),
            pltpu.VMEM((QPW * 2,), jnp.float32),
        ],
    )
    def knn(px_hbm, py_hbm, fl_hbm, fh_hbm, pn_hbm, out_hbm,
            px_v, py_v, fl_v, fh_v, pn_v, out_v):
        wid = lax.axis_index("s") * 2 + lax.axis_index("c")
        b = wid // WPB
        half = wid % WPB
        pltpu.sync_copy(px_hbm.at[b], px_v)
        pltpu.sync_copy(py_hbm.at[b], py_v)
        pltpu.sync_copy(fl_hbm.at[b], fl_v)
        pltpu.sync_copy(fh_hbm.at[b], fh_v)
        pltpu.sync_copy(pn_hbm, pn_v)
        lanes = jnp.arange(L, dtype=jnp.int32)
        pn_b = plsc.load_gather(pn_v, [jnp.zeros((L,), jnp.int32) + b])
        small = pn_b < K
        qbase = half * QPW

        QI = 4

        def group_body(g, carry):
            def q_body(j, acc):
                a0, a1 = acc
                q0 = qbase + g * L + j * QI
                qx = [(q0 + u) // W for u in range(QI)]
                qx = [v.astype(jnp.float32) for v in qx]
                qy = [((q0 + u) % W).astype(jnp.float32) for u in range(QI)]
                bk0 = jnp.full((L,), jnp.inf, jnp.float32)
                bv0 = jnp.full((L,), 2**30, jnp.int32)

                def chunk(t, c):
                    o = t * L
                    pxc = px_v[pl.ds(o, L)]
                    pyc = py_v[pl.ds(o, L)]
                    idxc = o + lanes
                    nxt = []
                    for u in range(QI):
                        bk, bv = c[2 * u], c[2 * u + 1]
                        dx = pxc - qx[u]
                        dy = pyc - qy[u]
                        d = dx * dx + dy * dy
                        ds_, is_ = plsc.sort_key_val(d, idxc, descending=True)
                        take = (ds_ < bk) | ((ds_ == bk) & (is_ < bv))
                        nk = jnp.where(take, ds_, bk)
                        nv = jnp.where(take, is_, bv)
                        nk, nv = plsc.sort_key_val(nk, nv)
                        nxt += [nk, nv]
                    return tuple(nxt)

                res = lax.fori_loop(0, N // L, chunk, (bk0, bv0) * QI)
                for u in range(QI):
                    bv = jnp.where(small, lanes, res[2 * u + 1])
                    f0 = plsc.load_gather(fl_v, [bv])
                    f1 = plsc.load_gather(fh_v, [bv])
                    s0 = jnp.sum(f0)
                    s1 = jnp.sum(f1)
                    c0 = jnp.sum(jnp.where(f0 != 0.0, 1.0, 0.0))
                    c1 = jnp.sum(jnp.where(f1 != 0.0, 1.0, 0.0))
                    c0 = jnp.where(c0 == 0.0, 1.0, c0)
                    c1 = jnp.where(c1 == 0.0, 1.0, c1)
                    av0 = jnp.broadcast_to(s0, (L,)) / jnp.broadcast_to(c0, (L,))
                    av1 = jnp.broadcast_to(s1, (L,)) / jnp.broadcast_to(c1, (L,))
                    sel = lanes == (j * QI + u)
                    a0 = jnp.where(sel, av0, a0)
                    a1 = jnp.where(sel, av1, a1)
                return a0, a1

            z = jnp.zeros((L,), jnp.float32)
            a0, a1 = lax.fori_loop(0, L // QI, q_body, (z, z))
            m = jnp.maximum(a0, a1)
            u0 = jnp.exp(a0 - m)
            u1 = jnp.exp(a1 - m)
            den = u0 + u1
            p0 = u0 / den
            p1 = u1 / den
            eq = p0 == p1
            p0 = jnp.where(eq, 1.0, p0)
            p1 = jnp.where(eq, 0.0, p1)
            i0 = g * (2 * L) + 2 * lanes
            plsc.store_scatter(out_v, [i0], p0)
            plsc.store_scatter(out_v, [i0 + 1], p1)
            return carry

        lax.fori_loop(0, NG, group_body, 0)
        pltpu.sync_copy(out_v, out_hbm.at[b, half])

    return knn(px, py, fl, fh, pn)


def kernel(coords, features, res, points_num):
    p = jnp.asarray(res, jnp.float32)
    cmax = jnp.max(coords, axis=-2, keepdims=True)
    cmin = jnp.min(coords, axis=-2, keepdims=True)
    center = (cmax + cmin) / 2
    scale = jnp.maximum(cmax - cmin, 1e-05) / 2
    cn = ((coords - center) / scale + 1) * 0.8 * p / 2 + 0.1 * p
    valid = jnp.arange(N)[None, :] < points_num[:, None]
    px = jnp.where(valid, cn[..., 0], 1e30)
    py = jnp.where(valid, cn[..., 1], 1e30)
    fl = jnp.minimum(features[..., 0], features[..., 1])
    fh = jnp.maximum(features[..., 0], features[..., 1])
    out = _sc_knn(px, py, fl, fh, points_num.astype(jnp.int32))
    return out.reshape(B, H, W, 2)


# speedup vs baseline: 80.0904x; 1.3716x over previous
import functools

import jax
import jax.numpy as jnp
from jax import lax
from jax.experimental import pallas as pl
from jax.experimental.pallas import tpu as pltpu
from jax.experimental.pallas import tpu_sc as plsc

H = 48
W = 48
S = H * W
N = 1024
B = 16
K = 16
L = 16
WPB = 2
QPW = S // WPB
NG = QPW // L


def _sc_knn(px, py, fl, fh, pn):
    mesh = plsc.VectorSubcoreMesh(core_axis_name="c", subcore_axis_name="s")

    @functools.partial(
        pl.kernel,
        out_type=jax.ShapeDtypeStruct((B, WPB, QPW * 2), jnp.float32),
        mesh=mesh,
        compiler_params=pltpu.CompilerParams(needs_layout_passes=False),
        scratch_types=[
            pltpu.VMEM((N,), jnp.float32),
            pltpu.VMEM((N,), jnp.float32),
            pltpu.VMEM((N,), jnp.float32),
            pltpu.VMEM((N,), jnp.float32),
            pltpu.VMEM((B,), jnp.int32),
            pltpu.VMEM((QPW * 2,), jnp.float32),
        ],
    )
    def knn(px_hbm, py_hbm, fl_hbm, fh_hbm, pn_hbm, out_hbm,
            px_v, py_v, fl_v, fh_v, pn_v, out_v):
        wid = lax.axis_index("s") * 2 + lax.axis_index("c")
        b = wid // WPB
        half = wid % WPB
        pltpu.sync_copy(px_hbm.at[b], px_v)
        pltpu.sync_copy(py_hbm.at[b], py_v)
        pltpu.sync_copy(fl_hbm.at[b], fl_v)
        pltpu.sync_copy(fh_hbm.at[b], fh_v)
        pltpu.sync_copy(pn_hbm, pn_v)
        lanes = jnp.arange(L, dtype=jnp.int32)
        pn_b = plsc.load_gather(pn_v, [jnp.zeros((L,), jnp.int32) + b])
        small = pn_b < K
        qbase = half * QPW

        QI = 8

        def group_body(g, carry):
            def q_body(j, acc):
                a0, a1 = acc
                q0 = qbase + g * L + j * QI
                qx = [(q0 + u) // W for u in range(QI)]
                qx = [v.astype(jnp.float32) for v in qx]
                qy = [((q0 + u) % W).astype(jnp.float32) for u in range(QI)]
                bk0 = jnp.full((L,), jnp.inf, jnp.float32)
                bv0 = jnp.full((L,), 2**30, jnp.int32)

                def chunk(t, c):
                    o = t * L
                    pxc = px_v[pl.ds(o, L)]
                    pyc = py_v[pl.ds(o, L)]
                    idxc = o + lanes
                    nxt = []
                    for u in range(QI):
                        bk, bv = c[2 * u], c[2 * u + 1]
                        dx = pxc - qx[u]
                        dy = pyc - qy[u]
                        d = dx * dx + dy * dy
                        ds_, is_ = plsc.sort_key_val(d, idxc, descending=True)
                        take = (ds_ < bk) | ((ds_ == bk) & (is_ < bv))
                        nk = jnp.where(take, ds_, bk)
                        nv = jnp.where(take, is_, bv)
                        nk, nv = plsc.sort_key_val(nk, nv)
                        nxt += [nk, nv]
                    return tuple(nxt)

                res = lax.fori_loop(0, N // L, chunk, (bk0, bv0) * QI)
                for u in range(QI):
                    bv = jnp.where(small, lanes, res[2 * u + 1])
                    f0 = plsc.load_gather(fl_v, [bv])
                    f1 = plsc.load_gather(fh_v, [bv])
                    s0 = jnp.sum(f0)
                    s1 = jnp.sum(f1)
                    c0 = jnp.sum(jnp.where(f0 != 0.0, 1.0, 0.0))
                    c1 = jnp.sum(jnp.where(f1 != 0.0, 1.0, 0.0))
                    c0 = jnp.where(c0 == 0.0, 1.0, c0)
                    c1 = jnp.where(c1 == 0.0, 1.0, c1)
                    av0 = jnp.broadcast_to(s0, (L,)) / jnp.broadcast_to(c0, (L,))
                    av1 = jnp.broadcast_to(s1, (L,)) / jnp.broadcast_to(c1, (L,))
                    sel = lanes == (j * QI + u)
                    a0 = jnp.where(sel, av0, a0)
                    a1 = jnp.where(sel, av1, a1)
                return a0, a1

            z = jnp.zeros((L,), jnp.float32)
            a0, a1 = lax.fori_loop(0, L // QI, q_body, (z, z))
            m = jnp.maximum(a0, a1)
            u0 = jnp.exp(a0 - m)
            u1 = jnp.exp(a1 - m)
            den = u0 + u1
            p0 = u0 / den
            p1 = u1 / den
            eq = p0 == p1
            p0 = jnp.where(eq, 1.0, p0)
            p1 = jnp.where(eq, 0.0, p1)
            i0 = g * (2 * L) + 2 * lanes
            plsc.store_scatter(out_v, [i0], p0)
            plsc.store_scatter(out_v, [i0 + 1], p1)
            return carry

        lax.fori_loop(0, NG, group_body, 0)
        pltpu.sync_copy(out_v, out_hbm.at[b, half])

    return knn(px, py, fl, fh, pn)


def kernel(coords, features, res, points_num):
    p = jnp.asarray(res, jnp.float32)
    cmax = jnp.max(coords, axis=-2, keepdims=True)
    cmin = jnp.min(coords, axis=-2, keepdims=True)
    center = (cmax + cmin) / 2
    scale = jnp.maximum(cmax - cmin, 1e-05) / 2
    cn = ((coords - center) / scale + 1) * 0.8 * p / 2 + 0.1 * p
    valid = jnp.arange(N)[None, :] < points_num[:, None]
    px = jnp.where(valid, cn[..., 0], 1e30)
    py = jnp.where(valid, cn[..., 1], 1e30)
    fl = jnp.minimum(features[..., 0], features[..., 1])
    fh = jnp.maximum(features[..., 0], features[..., 1])
    out = _sc_knn(px, py, fl, fh, points_num.astype(jnp.int32))
    return out.reshape(B, H, W, 2)


# speedup vs baseline: 96.7325x; 1.2078x over previous
import functools

import jax
import jax.numpy as jnp
from jax import lax
from jax.experimental import pallas as pl
from jax.experimental.pallas import tpu as pltpu
from jax.experimental.pallas import tpu_sc as plsc

H = 48
W = 48
S = H * W
N = 1024
B = 16
K = 16
L = 16
WPB = 2
QPW = S // WPB
NG = QPW // L


def _sc_knn(px, py, p2, fl, fh, pn):
    mesh = plsc.VectorSubcoreMesh(core_axis_name="c", subcore_axis_name="s")

    @functools.partial(
        pl.kernel,
        out_type=jax.ShapeDtypeStruct((B, WPB, QPW * 2), jnp.float32),
        mesh=mesh,
        compiler_params=pltpu.CompilerParams(needs_layout_passes=False),
        scratch_types=[
            pltpu.VMEM((N,), jnp.float32),
            pltpu.VMEM((N,), jnp.float32),
            pltpu.VMEM((N,), jnp.float32),
            pltpu.VMEM((N,), jnp.float32),
            pltpu.VMEM((N,), jnp.float32),
            pltpu.VMEM((B,), jnp.int32),
            pltpu.VMEM((QPW * 2,), jnp.float32),
        ],
    )
    def knn(px_hbm, py_hbm, p2_hbm, fl_hbm, fh_hbm, pn_hbm, out_hbm,
            px_v, py_v, p2_v, fl_v, fh_v, pn_v, out_v):
        wid = lax.axis_index("s") * 2 + lax.axis_index("c")
        b = wid // WPB
        half = wid % WPB
        pltpu.sync_copy(px_hbm.at[b], px_v)
        pltpu.sync_copy(py_hbm.at[b], py_v)
        pltpu.sync_copy(p2_hbm.at[b], p2_v)
        pltpu.sync_copy(fl_hbm.at[b], fl_v)
        pltpu.sync_copy(fh_hbm.at[b], fh_v)
        pltpu.sync_copy(pn_hbm, pn_v)
        lanes = jnp.arange(L, dtype=jnp.int32)
        pn_b = plsc.load_gather(pn_v, [jnp.zeros((L,), jnp.int32) + b])
        small = pn_b < K
        qbase = half * QPW

        QI = 8

        def group_body(g, carry):
            def q_body(j, acc):
                a0, a1 = acc
                q0 = qbase + g * L + j * QI
                tqx = [(2 * ((q0 + u) // W)).astype(jnp.float32) for u in range(QI)]
                tqy = [(2 * ((q0 + u) % W)).astype(jnp.float32) for u in range(QI)]
                bk0 = jnp.full((L,), jnp.inf, jnp.float32)
                bv0 = jnp.full((L,), 2**30, jnp.int32)

                def chunk(t, c):
                    o = t * L
                    pxc = px_v[pl.ds(o, L)]
                    pyc = py_v[pl.ds(o, L)]
                    p2c = p2_v[pl.ds(o, L)]
                    idxc = o + lanes
                    nxt = []
                    for u in range(QI):
                        bk, bv = c[2 * u], c[2 * u + 1]
                        d = p2c - tqx[u] * pxc - tqy[u] * pyc
                        ds_, is_ = plsc.sort_key_val(d, idxc, descending=True)
                        take = ds_ < bk
                        nk = jnp.where(take, ds_, bk)
                        nv = jnp.where(take, is_, bv)
                        nk, nv = plsc.sort_key_val(nk, nv)
                        nxt += [nk, nv]
                    return tuple(nxt)

                res = lax.fori_loop(0, N // L, chunk, (bk0, bv0) * QI)
                for u in range(QI):
                    bv = jnp.where(small, lanes, res[2 * u + 1])
                    f0 = plsc.load_gather(fl_v, [bv])
                    f1 = plsc.load_gather(fh_v, [bv])
                    s0 = jnp.sum(f0)
                    s1 = jnp.sum(f1)
                    c0 = jnp.sum(jnp.where(f0 != 0.0, 1.0, 0.0))
                    c1 = jnp.sum(jnp.where(f1 != 0.0, 1.0, 0.0))
                    c0 = jnp.where(c0 == 0.0, 1.0, c0)
                    c1 = jnp.where(c1 == 0.0, 1.0, c1)
                    av0 = jnp.broadcast_to(s0, (L,)) / jnp.broadcast_to(c0, (L,))
                    av1 = jnp.broadcast_to(s1, (L,)) / jnp.broadcast_to(c1, (L,))
                    sel = lanes == (j * QI + u)
                    a0 = jnp.where(sel, av0, a0)
                    a1 = jnp.where(sel, av1, a1)
                return a0, a1

            z = jnp.zeros((L,), jnp.float32)
            a0, a1 = lax.fori_loop(0, L // QI, q_body, (z, z))
            m = jnp.maximum(a0, a1)
            u0 = jnp.exp(a0 - m)
            u1 = jnp.exp(a1 - m)
            den = u0 + u1
            p0 = u0 / den
            p1 = u1 / den
            eq = p0 == p1
            p0 = jnp.where(eq, 1.0, p0)
            p1 = jnp.where(eq, 0.0, p1)
            i0 = g * (2 * L) + 2 * lanes
            plsc.store_scatter(out_v, [i0], p0)
            plsc.store_scatter(out_v, [i0 + 1], p1)
            return carry

        lax.fori_loop(0, NG, group_body, 0)
        pltpu.sync_copy(out_v, out_hbm.at[b, half])

    return knn(px, py, p2, fl, fh, pn)


def kernel(coords, features, res, points_num):
    p = jnp.asarray(res, jnp.float32)
    cmax = jnp.max(coords, axis=-2, keepdims=True)
    cmin = jnp.min(coords, axis=-2, keepdims=True)
    center = (cmax + cmin) / 2
    scale = jnp.maximum(cmax - cmin, 1e-05) / 2
    cn = ((coords - center) / scale + 1) * 0.8 * p / 2 + 0.1 * p
    valid = jnp.arange(N)[None, :] < points_num[:, None]
    px = jnp.where(valid, cn[..., 0], 1e30)
    py = jnp.where(valid, cn[..., 1], 1e30)
    p2 = px * px + py * py
    fl = jnp.minimum(features[..., 0], features[..., 1])
    fh = jnp.maximum(features[..., 0], features[..., 1])
    out = _sc_knn(px, py, p2, fl, fh, points_num.astype(jnp.int32))
    return out.reshape(B, H, W, 2)


# speedup vs baseline: 98.7536x; 1.0209x over previous
import functools

import jax
import jax.numpy as jnp
from jax import lax
from jax.experimental import pallas as pl
from jax.experimental.pallas import tpu as pltpu
from jax.experimental.pallas import tpu_sc as plsc

H = 48
W = 48
S = H * W
N = 1024
B = 16
K = 16
L = 16
WPB = 2
QPW = S // WPB
NG = QPW // L


def _sc_knn(px, py, p2, fl, fh, pn):
    mesh = plsc.VectorSubcoreMesh(core_axis_name="c", subcore_axis_name="s")

    @functools.partial(
        pl.kernel,
        out_type=jax.ShapeDtypeStruct((B, WPB, QPW * 2), jnp.float32),
        mesh=mesh,
        compiler_params=pltpu.CompilerParams(needs_layout_passes=False),
        scratch_types=[
            pltpu.VMEM((N,), jnp.float32),
            pltpu.VMEM((N,), jnp.float32),
            pltpu.VMEM((N,), jnp.float32),
            pltpu.VMEM((N,), jnp.float32),
            pltpu.VMEM((N,), jnp.float32),
            pltpu.VMEM((B,), jnp.int32),
            pltpu.VMEM((QPW * 2,), jnp.float32),
        ],
    )
    def knn(px_hbm, py_hbm, p2_hbm, fl_hbm, fh_hbm, pn_hbm, out_hbm,
            px_v, py_v, p2_v, fl_v, fh_v, pn_v, out_v):
        wid = lax.axis_index("s") * 2 + lax.axis_index("c")
        b = wid // WPB
        half = wid % WPB
        pltpu.sync_copy(px_hbm.at[b], px_v)
        pltpu.sync_copy(py_hbm.at[b], py_v)
        pltpu.sync_copy(p2_hbm.at[b], p2_v)
        pltpu.sync_copy(fl_hbm.at[b], fl_v)
        pltpu.sync_copy(fh_hbm.at[b], fh_v)
        pltpu.sync_copy(pn_hbm, pn_v)
        lanes = jnp.arange(L, dtype=jnp.int32)
        pn_b = plsc.load_gather(pn_v, [jnp.zeros((L,), jnp.int32) + b])
        small = pn_b < K
        qbase = half * QPW

        QI = 16

        def group_body(g, carry):
            def q_body(j, acc):
                a0, a1 = acc
                q0 = qbase + g * L + j * QI
                tqx = [(2 * ((q0 + u) // W)).astype(jnp.float32) for u in range(QI)]
                tqy = [(2 * ((q0 + u) % W)).astype(jnp.float32) for u in range(QI)]
                bk0 = jnp.full((L,), jnp.inf, jnp.float32)
                bv0 = jnp.full((L,), 2**30, jnp.int32)

                def chunk(t, c):
                    o = t * L
                    pxc = px_v[pl.ds(o, L)]
                    pyc = py_v[pl.ds(o, L)]
                    p2c = p2_v[pl.ds(o, L)]
                    idxc = o + lanes
                    nxt = []
                    for u in range(QI):
                        bk, bv = c[2 * u], c[2 * u + 1]
                        d = p2c - tqx[u] * pxc - tqy[u] * pyc
                        ds_, is_ = plsc.sort_key_val(d, idxc, descending=True)
                        take = ds_ < bk
                        nk = jnp.where(take, ds_, bk)
                        nv = jnp.where(take, is_, bv)
                        nk, nv = plsc.sort_key_val(nk, nv)
                        nxt += [nk, nv]
                    return tuple(nxt)

                res = lax.fori_loop(0, N // L, chunk, (bk0, bv0) * QI)
                for u in range(QI):
                    bv = jnp.where(small, lanes, res[2 * u + 1])
                    f0 = plsc.load_gather(fl_v, [bv])
                    f1 = plsc.load_gather(fh_v, [bv])
                    s0 = jnp.sum(f0)
                    s1 = jnp.sum(f1)
                    c0 = jnp.sum(jnp.where(f0 != 0.0, 1.0, 0.0))
                    c1 = jnp.sum(jnp.where(f1 != 0.0, 1.0, 0.0))
                    c0 = jnp.where(c0 == 0.0, 1.0, c0)
                    c1 = jnp.where(c1 == 0.0, 1.0, c1)
                    av0 = jnp.broadcast_to(s0, (L,)) / jnp.broadcast_to(c0, (L,))
                    av1 = jnp.broadcast_to(s1, (L,)) / jnp.broadcast_to(c1, (L,))
                    sel = lanes == (j * QI + u)
                    a0 = jnp.where(sel, av0, a0)
                    a1 = jnp.where(sel, av1, a1)
                return a0, a1

            z = jnp.zeros((L,), jnp.float32)
            a0, a1 = lax.fori_loop(0, L // QI, q_body, (z, z))
            m = jnp.maximum(a0, a1)
            u0 = jnp.exp(a0 - m)
            u1 = jnp.exp(a1 - m)
            den = u0 + u1
            p0 = u0 / den
            p1 = u1 / den
            eq = p0 == p1
            p0 = jnp.where(eq, 1.0, p0)
            p1 = jnp.where(eq, 0.0, p1)
            i0 = g * (2 * L) + 2 * lanes
            plsc.store_scatter(out_v, [i0], p0)
            plsc.store_scatter(out_v, [i0 + 1], p1)
            return carry

        lax.fori_loop(0, NG, group_body, 0)
        pltpu.sync_copy(out_v, out_hbm.at[b, half])

    return knn(px, py, p2, fl, fh, pn)


def kernel(coords, features, res, points_num):
    p = jnp.asarray(res, jnp.float32)
    cmax = jnp.max(coords, axis=-2, keepdims=True)
    cmin = jnp.min(coords, axis=-2, keepdims=True)
    center = (cmax + cmin) / 2
    scale = jnp.maximum(cmax - cmin, 1e-05) / 2
    cn = ((coords - center) / scale + 1) * 0.8 * p / 2 + 0.1 * p
    valid = jnp.arange(N)[None, :] < points_num[:, None]
    px = jnp.where(valid, cn[..., 0], 1e30)
    py = jnp.where(valid, cn[..., 1], 1e30)
    p2 = px * px + py * py
    fl = jnp.minimum(features[..., 0], features[..., 1])
    fh = jnp.maximum(features[..., 0], features[..., 1])
    out = _sc_knn(px, py, p2, fl, fh, points_num.astype(jnp.int32))
    return out.reshape(B, H, W, 2)


# speedup vs baseline: 98.8704x; 1.0012x over previous
import functools

import jax
import jax.numpy as jnp
from jax import lax
from jax.experimental import pallas as pl
from jax.experimental.pallas import tpu as pltpu
from jax.experimental.pallas import tpu_sc as plsc

H = 48
W = 48
S = H * W
N = 1024
B = 16
K = 16
L = 16
WPB = 2
QPW = S // WPB
NG = QPW // L


def _sc_knn(px, py, p2, fl, fh, pn):
    mesh = plsc.VectorSubcoreMesh(core_axis_name="c", subcore_axis_name="s")

    @functools.partial(
        pl.kernel,
        out_type=jax.ShapeDtypeStruct((B, WPB, QPW * 2), jnp.float32),
        mesh=mesh,
        compiler_params=pltpu.CompilerParams(needs_layout_passes=False),
        scratch_types=[
            pltpu.VMEM((N,), jnp.float32),
            pltpu.VMEM((N,), jnp.float32),
            pltpu.VMEM((N,), jnp.float32),
            pltpu.VMEM((N,), jnp.float32),
            pltpu.VMEM((N,), jnp.float32),
            pltpu.VMEM((B,), jnp.int32),
            pltpu.VMEM((QPW * 2,), jnp.float32),
        ],
    )
    def knn(px_hbm, py_hbm, p2_hbm, fl_hbm, fh_hbm, pn_hbm, out_hbm,
            px_v, py_v, p2_v, fl_v, fh_v, pn_v, out_v):
        wid = lax.axis_index("s") * 2 + lax.axis_index("c")
        b = wid // WPB
        half = wid % WPB
        pltpu.sync_copy(px_hbm.at[b], px_v)
        pltpu.sync_copy(py_hbm.at[b], py_v)
        pltpu.sync_copy(p2_hbm.at[b], p2_v)
        pltpu.sync_copy(fl_hbm.at[b], fl_v)
        pltpu.sync_copy(fh_hbm.at[b], fh_v)
        pltpu.sync_copy(pn_hbm, pn_v)
        lanes = jnp.arange(L, dtype=jnp.int32)
        pn_b = plsc.load_gather(pn_v, [jnp.zeros((L,), jnp.int32) + b])
        small = pn_b < K
        qbase = half * QPW

        QI = 16

        def group_body(g, carry):
            def q_body(j, acc):
                a0, a1 = acc
                q0 = qbase + g * L + j * QI
                tqx = (2 * (q0 // W)).astype(jnp.float32)
                tqy = [(2 * ((q0 + u) % W)).astype(jnp.float32) for u in range(QI)]
                bk0 = jnp.full((L,), jnp.inf, jnp.float32)
                bv0 = jnp.full((L,), 2**30, jnp.int32)

                def chunk(t, c):
                    o = t * L
                    pxc = px_v[pl.ds(o, L)]
                    pyc = py_v[pl.ds(o, L)]
                    p2c = p2_v[pl.ds(o, L)]
                    idxc = o + lanes
                    ax = p2c - tqx * pxc
                    nxt = []
                    for u in range(QI):
                        bk, bv = c[2 * u], c[2 * u + 1]
                        d = ax - tqy[u] * pyc
                        ds_, is_ = plsc.sort_key_val(d, idxc, descending=True)
                        take = ds_ < bk
                        nk = jnp.where(take, ds_, bk)
                        nv = jnp.where(take, is_, bv)
                        nk, nv = plsc.sort_key_val(nk, nv)
                        nxt += [nk, nv]
                    return tuple(nxt)

                res = lax.fori_loop(0, N // L, chunk, (bk0, bv0) * QI)
                for u in range(QI):
                    bv = jnp.where(small, lanes, res[2 * u + 1])
                    f0 = plsc.load_gather(fl_v, [bv])
                    f1 = plsc.load_gather(fh_v, [bv])
                    s0 = jnp.sum(f0)
                    s1 = jnp.sum(f1)
                    c0 = jnp.sum(jnp.where(f0 != 0.0, 1.0, 0.0))
                    c1 = jnp.sum(jnp.where(f1 != 0.0, 1.0, 0.0))
                    c0 = jnp.where(c0 == 0.0, 1.0, c0)
                    c1 = jnp.where(c1 == 0.0, 1.0, c1)
                    av0 = jnp.broadcast_to(s0, (L,)) / jnp.broadcast_to(c0, (L,))
                    av1 = jnp.broadcast_to(s1, (L,)) / jnp.broadcast_to(c1, (L,))
                    sel = lanes == (j * QI + u)
                    a0 = jnp.where(sel, av0, a0)
                    a1 = jnp.where(sel, av1, a1)
                return a0, a1

            z = jnp.zeros((L,), jnp.float32)
            a0, a1 = lax.fori_loop(0, L // QI, q_body, (z, z))
            m = jnp.maximum(a0, a1)
            u0 = jnp.exp(a0 - m)
            u1 = jnp.exp(a1 - m)
            den = u0 + u1
            p0 = u0 / den
            p1 = u1 / den
            eq = p0 == p1
            p0 = jnp.where(eq, 1.0, p0)
            p1 = jnp.where(eq, 0.0, p1)
            i0 = g * (2 * L) + 2 * lanes
            plsc.store_scatter(out_v, [i0], p0)
            plsc.store_scatter(out_v, [i0 + 1], p1)
            return carry

        lax.fori_loop(0, NG, group_body, 0)
        pltpu.sync_copy(out_v, out_hbm.at[b, half])

    return knn(px, py, p2, fl, fh, pn)


def kernel(coords, features, res, points_num):
    p = jnp.asarray(res, jnp.float32)
    cmax = jnp.max(coords, axis=-2, keepdims=True)
    cmin = jnp.min(coords, axis=-2, keepdims=True)
    center = (cmax + cmin) / 2
    scale = jnp.maximum(cmax - cmin, 1e-05) / 2
    cn = ((coords - center) / scale + 1) * 0.8 * p / 2 + 0.1 * p
    valid = jnp.arange(N)[None, :] < points_num[:, None]
    px = jnp.where(valid, cn[..., 0], 1e30)
    py = jnp.where(valid, cn[..., 1], 1e30)
    p2 = px * px + py * py
    fl = jnp.minimum(features[..., 0], features[..., 1])
    fh = jnp.maximum(features[..., 0], features[..., 1])
    out = _sc_knn(px, py, p2, fl, fh, points_num.astype(jnp.int32))
    return out.reshape(B, H, W, 2)


# speedup vs baseline: 123.8276x; 1.2524x over previous
import functools

import jax
import jax.numpy as jnp
from jax import lax
from jax.experimental import pallas as pl
from jax.experimental.pallas import tpu as pltpu
from jax.experimental.pallas import tpu_sc as plsc

H = 48
W = 48
S = H * W
N = 1024
B = 16
K = 16
L = 16
NCH = N // L
WPB = 2
QPW = S // WPB
RPW = H // WPB
TI = RPW // 4
TJ = W // 4
NT = TI * TJ
INF = float("inf")


def _sc_knn(pxs, pys, p2s, oidx, ybnd, fl, fh, pn):
    mesh = plsc.VectorSubcoreMesh(core_axis_name="c", subcore_axis_name="s")

    @functools.partial(
        pl.kernel,
        out_type=jax.ShapeDtypeStruct((B, WPB, QPW * 2), jnp.float32),
        mesh=mesh,
        compiler_params=pltpu.CompilerParams(needs_layout_passes=False),
        scratch_types=[
            pltpu.VMEM((N,), jnp.float32),
            pltpu.VMEM((N,), jnp.float32),
            pltpu.VMEM((N,), jnp.float32),
            pltpu.VMEM((N,), jnp.int32),
            pltpu.VMEM((2 * NCH,), jnp.float32),
            pltpu.VMEM((N,), jnp.float32),
            pltpu.VMEM((N,), jnp.float32),
            pltpu.VMEM((B,), jnp.int32),
            pltpu.VMEM((QPW * 2,), jnp.float32),
        ],
    )
    def knn(pxs_hbm, pys_hbm, p2s_hbm, oidx_hbm, ybnd_hbm, fl_hbm, fh_hbm,
            pn_hbm, out_hbm,
            pxs_v, pys_v, p2s_v, oidx_v, ybnd_v, fl_v, fh_v, pn_v, out_v):
        wid = lax.axis_index("s") * 2 + lax.axis_index("c")
        b = wid // WPB
        half = wid % WPB
        pltpu.sync_copy(pxs_hbm.at[b], pxs_v)
        pltpu.sync_copy(pys_hbm.at[b], pys_v)
        pltpu.sync_copy(p2s_hbm.at[b], p2s_v)
        pltpu.sync_copy(oidx_hbm.at[b], oidx_v)
        pltpu.sync_copy(ybnd_hbm.at[b], ybnd_v)
        pltpu.sync_copy(fl_hbm.at[b], fl_v)
        pltpu.sync_copy(fh_hbm.at[b], fh_v)
        pltpu.sync_copy(pn_hbm, pn_v)
        lanes = jnp.arange(L, dtype=jnp.int32)
        pn_b = plsc.load_gather(pn_v, [jnp.zeros((L,), jnp.int32) + b])
        small = pn_b < K
        row0 = half * RPW

        def tile_body(tt, carry):
            ti = tt // TJ
            tj = tt % TJ
            i0 = row0 + ti * 4
            j0 = tj * 4
            tqx = [(2 * (i0 + v)).astype(jnp.float32) for v in range(4)]
            tqy = [(2 * (j0 + v)).astype(jnp.float32) for v in range(4)]
            qylo = j0.astype(jnp.float32)
            qyhi = qylo + 3.0

            def _merge16(pxc, pyc, p2c, idxc, bks, bvs):
                ax = [p2c - tqx[v] * pxc for v in range(4)]
                by = [tqy[v] * pyc for v in range(4)]
                nk, nv = [], []
                for u in range(L):
                    d = ax[u // 4] - by[u % 4]
                    ds_, is_ = plsc.sort_key_val(d, idxc, descending=True)
                    take = ds_ < bks[u]
                    mk = jnp.where(take, ds_, bks[u])
                    mv = jnp.where(take, is_, bvs[u])
                    mk, mv = plsc.sort_key_val(mk, mv)
                    nk.append(mk)
                    nv.append(mv)
                return nk, nv

            def scan(t, bks, bvs):
                o = t * L
                pxc = pxs_v[pl.ds(o, L)]
                pyc = pys_v[pl.ds(o, L)]
                p2c = p2s_v[pl.ds(o, L)]
                return _merge16(pxc, pyc, p2c, o + lanes, bks, bvs)

            def bounds(t_dn, t_up):
                yb_dn = ybnd_v[pl.ds(t_dn, L)][0]
                yb_up = ybnd_v[pl.ds(t_up + 1, L)][0]
                g_dn = jnp.maximum(qylo - yb_dn, 0.0)
                g_up = jnp.maximum(yb_up - qyhi, 0.0)
                b_dn = jnp.where(t_dn > 0, g_dn * g_dn, INF)
                b_up = jnp.where(t_up < NCH - 1, g_up * g_up, INF)
                return b_dn, b_up

            def gmax_of(bks):
                m = bks[0]
                for u in range(1, L):
                    m = jnp.maximum(m, bks[u])
                return jnp.max(m)

            cy = qylo + 1.5
            acc = jnp.zeros((L,), jnp.int32)
            for v in range(NCH // L):
                yc = ybnd_v[pl.ds(v * L, L)]
                acc = acc + jnp.where(yc <= cy, 1, 0)
            t0 = jnp.clip(jnp.sum(acc) - 1, 0, NCH - 1)

            bk0 = [jnp.full((L,), INF, jnp.float32) for _ in range(L)]
            bv0 = [jnp.full((L,), 2**30, jnp.int32) for _ in range(L)]
            bks, bvs = scan(t0, bk0, bv0)
            b_dn, b_up = bounds(t0, t0)
            cont = gmax_of(bks) > jnp.minimum(b_dn, b_up)

            def w_cond(st):
                return st[2]

            def w_body(st):
                t_dn, t_up, _ = st[0], st[1], st[2]
                bks = list(st[3:3 + L])
                bvs = list(st[3 + L:3 + 2 * L])
                b_dn, b_up = bounds(t_dn, t_up)
                go_dn = b_dn <= b_up
                t_next = jnp.where(go_dn, t_dn - 1, t_up + 1)
                bks, bvs = scan(t_next, bks, bvs)
                t_dn = jnp.where(go_dn, t_next, t_dn)
                t_up = jnp.where(go_dn, t_up, t_next)
                b_dn, b_up = bounds(t_dn, t_up)
                cont = gmax_of(bks) > jnp.minimum(b_dn, b_up)
                return (t_dn, t_up, cont, *bks, *bvs)

            st = lax.while_loop(w_cond, w_body, (t0, t0, cont, *bks, *bvs))
            bks = list(st[3:3 + L])
            bvs = list(st[3 + L:3 + 2 * L])

            a0 = jnp.zeros((L,), jnp.float32)
            a1 = jnp.zeros((L,), jnp.float32)
            for u in range(L):
                orig = plsc.load_gather(oidx_v, [bvs[u]])
                orig = jnp.where(small, lanes, orig)
                f0 = plsc.load_gather(fl_v, [orig])
                f1 = plsc.load_gather(fh_v, [orig])
                s0 = jnp.sum(f0)
                s1 = jnp.sum(f1)
                c0 = jnp.sum(jnp.where(f0 != 0.0, 1.0, 0.0))
                c1 = jnp.sum(jnp.where(f1 != 0.0, 1.0, 0.0))
                c0 = jnp.where(c0 == 0.0, 1.0, c0)
                c1 = jnp.where(c1 == 0.0, 1.0, c1)
                av0 = jnp.broadcast_to(s0, (L,)) / jnp.broadcast_to(c0, (L,))
                av1 = jnp.broadcast_to(s1, (L,)) / jnp.broadcast_to(c1, (L,))
                sel = lanes == u
                a0 = jnp.where(sel, av0, a0)
                a1 = jnp.where(sel, av1, a1)

            m = jnp.maximum(a0, a1)
            u0 = jnp.exp(a0 - m)
            u1 = jnp.exp(a1 - m)
            den = u0 + u1
            p0 = u0 / den
            p1 = u1 / den
            eq = p0 == p1
            p0 = jnp.where(eq, 1.0, p0)
            p1 = jnp.where(eq, 0.0, p1)
            lq = (ti * 4 + lanes // 4) * W + j0 + lanes % 4
            plsc.store_scatter(out_v, [2 * lq], p0)
            plsc.store_scatter(out_v, [2 * lq + 1], p1)
            return carry

        lax.fori_loop(0, NT, tile_body, 0)
        pltpu.sync_copy(out_v, out_hbm.at[b, half])

    return knn(pxs, pys, p2s, oidx, ybnd, fl, fh, pn)


def kernel(coords, features, res, points_num):
    p = jnp.asarray(res, jnp.float32)
    cmax = jnp.max(coords, axis=-2, keepdims=True)
    cmin = jnp.min(coords, axis=-2, keepdims=True)
    center = (cmax + cmin) / 2
    scale = jnp.maximum(cmax - cmin, 1e-05) / 2
    cn = ((coords - center) / scale + 1) * 0.8 * p / 2 + 0.1 * p
    valid = jnp.arange(N)[None, :] < points_num[:, None]
    px = jnp.where(valid, cn[..., 0], 1e30)
    py = jnp.where(valid, cn[..., 1], 1e30)
    perm = jnp.argsort(py, axis=1, stable=True)
    pxs = jnp.take_along_axis(px, perm, axis=1)
    pys = jnp.take_along_axis(py, perm, axis=1)
    p2s = pxs * pxs + pys * pys
    ybnd = jnp.concatenate(
        [pys[:, ::L], jnp.full((B, NCH), jnp.inf, jnp.float32)], axis=1)
    fl = jnp.minimum(features[..., 0], features[..., 1])
    fh = jnp.maximum(features[..., 0], features[..., 1])
    out = _sc_knn(pxs, pys, p2s, perm.astype(jnp.int32), ybnd, fl, fh,
                  points_num.astype(jnp.int32))
    return out.reshape(B, H, W, 2)
